# initial kernel scaffold (unmeasured)
import jax
import jax.numpy as jnp
from jax import lax
from jax.experimental import pallas as pl
from jax.experimental.pallas import tpu as pltpu

N_DEV = 8
BLOCK_M = 512


def kernel(x, dy, gamma):
    m, d = x.shape
    n_blocks = m // BLOCK_M

    def body(x_ref, dy_ref, out_ref, acc_ref, comm_ref, send_sems, recv_sems):
        step = pl.program_id(0)
        my = lax.axis_index("i")

        xb = x_ref[...]
        dyb = dy_ref[...]
        mu = jnp.mean(xb, axis=1, keepdims=True)
        xc = xb - mu
        var = jnp.mean(xc * xc, axis=1, keepdims=True)
        rstd = lax.rsqrt(var + 1e-5)
        xhat = xc * rstd
        part = jnp.concatenate(
            [
                jnp.sum(dyb * xhat, axis=0, keepdims=True),
                jnp.sum(dyb, axis=0, keepdims=True),
            ],
            axis=0,
        )

        @pl.when(step == 0)
        def _():
            acc_ref[...] = part

        @pl.when(step > 0)
        def _():
            acc_ref[...] = acc_ref[...] + part

        @pl.when(step == n_blocks - 1)
        def _():
            rdmas = []
            for k in range(1, N_DEV):
                rdma = pltpu.make_async_remote_copy(
                    src_ref=acc_ref,
                    dst_ref=comm_ref.at[k - 1],
                    send_sem=send_sems.at[k - 1],
                    recv_sem=recv_sems.at[k - 1],
                    device_id=(lax.rem(my + k, N_DEV),),
                    device_id_type=pl.DeviceIdType.MESH,
                )
                rdma.start()
                rdmas.append(rdma)
            total = acc_ref[...]
            for k in range(1, N_DEV):
                rdmas[k - 1].wait_recv()
                total = total + comm_ref[k - 1]
            out_ref[...] = total
            for r in rdmas:
                r.wait_send()

    return pl.pallas_call(
        body,
        grid=(n_blocks,),
        out_shape=jax.ShapeDtypeStruct((2, d), jnp.float32),
        in_specs=[
            pl.BlockSpec((BLOCK_M, d), lambda i: (i, 0)),
            pl.BlockSpec((BLOCK_M, d), lambda i: (i, 0)),
        ],
        out_specs=pl.BlockSpec((2, d), lambda i: (0, 0)),
        scratch_shapes=[
            pltpu.VMEM((2, d), jnp.float32),
            pltpu.VMEM((N_DEV - 1, 2, d), jnp.float32),
            pltpu.SemaphoreType.DMA((N_DEV - 1,)),
            pltpu.SemaphoreType.DMA((N_DEV - 1,)),
        ],
        compiler_params=pltpu.CompilerParams(
            dimension_semantics=("arbitrary",),
            collective_id=0,
        ),
    )(x, dy)


# baseline (device time: 33722 ns/iter reference)
import jax
import jax.numpy as jnp
from jax import lax
from jax.experimental import pallas as pl
from jax.experimental.pallas import tpu as pltpu

N_DEV = 8
BLOCK_M = 512


def kernel(x, dy, gamma):
    m, d = x.shape
    n_blocks = m // BLOCK_M

    def body(x_ref, dy_ref, out_ref, acc_ref, comm_ref, send_sems, recv_sems):
        step = pl.program_id(0)
        my = lax.axis_index("i")

        xb = x_ref[...]
        dyb = dy_ref[...]
        mu = jnp.mean(xb, axis=1, keepdims=True)
        xc = xb - mu
        var = jnp.mean(xc * xc, axis=1, keepdims=True)
        rstd = lax.rsqrt(var + 1e-5)
        xhat = xc * rstd
        part = jnp.concatenate(
            [
                jnp.sum(dyb * xhat, axis=0, keepdims=True),
                jnp.sum(dyb, axis=0, keepdims=True),
            ],
            axis=0,
        )

        @pl.when(step == 0)
        def _():
            acc_ref[...] = part

        @pl.when(step > 0)
        def _():
            acc_ref[...] = acc_ref[...] + part

        @pl.when(step == n_blocks - 1)
        def _():
            rdmas = []
            for k in range(1, N_DEV):
                rdma = pltpu.make_async_remote_copy(
                    src_ref=acc_ref,
                    dst_ref=comm_ref.at[k - 1],
                    send_sem=send_sems.at[k - 1],
                    recv_sem=recv_sems.at[k - 1],
                    device_id=(lax.rem(my + k, N_DEV),),
                    device_id_type=pl.DeviceIdType.MESH,
                )
                rdma.start()
                rdmas.append(rdma)
            total = acc_ref[...]
            for k in range(1, N_DEV):
                rdmas[k - 1].wait_recv()
                total = total + comm_ref[k - 1]
            out_ref[...] = total
            for r in rdmas:
                r.wait_send()

    return pl.pallas_call(
        body,
        grid=(n_blocks,),
        out_shape=jax.ShapeDtypeStruct((2, d), jnp.float32),
        in_specs=[
            pl.BlockSpec((BLOCK_M, d), lambda i: (i, 0)),
            pl.BlockSpec((BLOCK_M, d), lambda i: (i, 0)),
        ],
        out_specs=pl.BlockSpec((2, d), lambda i: (0, 0)),
        scratch_shapes=[
            pltpu.VMEM((2, d), jnp.float32),
            pltpu.VMEM((N_DEV - 1, 2, d), jnp.float32),
            pltpu.SemaphoreType.DMA((N_DEV - 1,)),
            pltpu.SemaphoreType.DMA((N_DEV - 1,)),
        ],
        compiler_params=pltpu.CompilerParams(
            dimension_semantics=("arbitrary",),
        ),
    )(x, dy)
